# baseline (speedup 1.0000x reference)
import functools

import jax
import jax.numpy as jnp
from jax import lax
from jax.experimental import pallas as pl
from jax.experimental.pallas import tpu as pltpu
from jax.experimental.pallas import tpu_sc as plsc

_B = 1024
_S = 200
_E = 64
_R = 1000
_RP = 1024

_HALF = 100
_NW = 32
_ROWS_W = _B // _NW
_HALVES_W = _ROWS_W * 2


def _sc_gather_sum(tpair, pair_idx, mrep):
    mesh = plsc.VectorSubcoreMesh(core_axis_name="c", subcore_axis_name="s")

    nbuf = 2

    @functools.partial(
        pl.kernel,
        out_type=(
            jax.ShapeDtypeStruct((_B, _S, _E), jnp.float32),
            jax.ShapeDtypeStruct((_B, _E), jnp.float32),
        ),
        mesh=mesh,
        scratch_types=[
            pltpu.VMEM((_HALVES_W, _HALF), jnp.int32),
            [pltpu.VMEM((_HALF, 128), jnp.float32) for _ in range(nbuf)],
            [pltpu.VMEM((_HALF, 16), jnp.float32) for _ in range(nbuf)],
            [pltpu.VMEM((_S, _E), jnp.float32) for _ in range(2)],
            pltpu.VMEM((_ROWS_W, _E), jnp.float32),
            [pltpu.SemaphoreType.DMA for _ in range(nbuf)],
            [pltpu.SemaphoreType.DMA for _ in range(nbuf)],
            [pltpu.SemaphoreType.DMA for _ in range(2)],
        ],
    )
    def k(tpair_hbm, pidx_hbm, mrep_hbm, emb_out, sums_out,
          pidx_v, bufs, mbufs, cbufs, sums_v, gsems, msems, csems):
        cid = lax.axis_index("c")
        sid = lax.axis_index("s")
        wid = sid * 2 + cid
        base_half = wid * _HALVES_W
        base_row = wid * _ROWS_W

        pltpu.sync_copy(pidx_hbm.at[pl.ds(base_half, _HALVES_W)], pidx_v)

        zero = jnp.zeros((16,), jnp.float32)

        def fire_gather(h, b):
            pltpu.async_copy(tpair_hbm.at[pidx_v.at[h]], bufs[b], gsems[b])
            pltpu.async_copy(mrep_hbm.at[base_half + h], mbufs[b], msems[b])

        def wait_gather(h, b):
            pltpu.make_async_copy(
                tpair_hbm.at[pidx_v.at[h]], bufs[b], gsems[b]).wait()
            pltpu.make_async_copy(
                mrep_hbm.at[base_half + h], mbufs[b], msems[b]).wait()

        def acc_compact(buf, mbuf, cbuf, s_base, acc):
            @pl.loop(0, _HALF, init_carry=acc, unroll=4)
            def inner(s, acc):
                a0, a1, a2, a3 = acc
                m = mbuf[s]
                out = []
                for c in range(4):
                    va = buf[s, pl.ds(c * 16, 16)]
                    vb = buf[s, pl.ds(64 + c * 16, 16)]
                    v = vb + (va - vb) * m
                    cbuf[s_base + s, pl.ds(c * 16, 16)] = v
                    out.append(v)
                return (a0 + out[0], a1 + out[1], a2 + out[2], a3 + out[3])
            return inner

        fire_gather(0, 0)

        @pl.loop(0, _HALVES_W, step=4)
        def outer(g0):
            acc = None
            for b in range(4):
                g = g0 + b
                r = (g0 + b) // 2
                cb = b // 2
                bi = b % 2
                wait_gather(g, bi)
                @pl.when(g + 1 < _HALVES_W)
                def _():
                    fire_gather(g + 1, (bi + 1) % 2)
                if b % 2 == 0:
                    @pl.when(g - 4 >= 0)
                    def _():
                        pltpu.make_async_copy(
                            cbufs[cb], emb_out.at[base_row + (g - 4) // 2],
                            csems[cb]).wait()
                    acc = acc_compact(bufs[bi], mbufs[bi], cbufs[cb], 0,
                                      (zero, zero, zero, zero))
                else:
                    acc = acc_compact(bufs[bi], mbufs[bi], cbufs[cb], _HALF, acc)
                    sums_v[r, pl.ds(0, 16)] = acc[0]
                    sums_v[r, pl.ds(16, 16)] = acc[1]
                    sums_v[r, pl.ds(32, 16)] = acc[2]
                    sums_v[r, pl.ds(48, 16)] = acc[3]
                    pltpu.async_copy(cbufs[cb], emb_out.at[base_row + r],
                                     csems[cb])

        for cb in range(2):
            pltpu.make_async_copy(
                cbufs[cb], emb_out.at[base_row + _ROWS_W - 2 + cb],
                csems[cb]).wait()

        pltpu.sync_copy(sums_v, sums_out.at[pl.ds(base_row, _ROWS_W)])

    return k(tpair, pair_idx, mrep)


def _mlp_body(s_ref, w1_ref, b1_ref, w2_ref, b2_ref, o_ref):
    x = s_ref[...] * (1.0 / _S)
    h = lax.dot_general(x, w1_ref[...], (((1,), (1,)), ((), ())),
                        preferred_element_type=jnp.float32) + b1_ref[...]
    h = jnp.maximum(h, 0.0)
    o_ref[...] = lax.dot_general(h, w2_ref[...], (((1,), (1,)), ((), ())),
                                 preferred_element_type=jnp.float32) + b2_ref[...]


def _mlp(sums, W1, b1, W2p, b2p):
    return pl.pallas_call(
        _mlp_body,
        out_shape=jax.ShapeDtypeStruct((_B, _RP), jnp.float32),
    )(sums, W1, b1.reshape(1, _E), W2p, b2p.reshape(1, _RP))


def kernel(atom_ids, atom_table, W1, b1, W2, b2):
    ids = atom_ids.astype(jnp.int32)
    tpair = atom_table.reshape(500000, 128)
    pair_idx = (ids >> 1).reshape(2 * _B, _HALF)
    meven = ((ids & 1) == 0).astype(jnp.float32).reshape(2 * _B, _HALF)
    mrep = jnp.broadcast_to(meven[:, :, None], (2 * _B, _HALF, 16))
    emb, sums = _sc_gather_sum(tpair, pair_idx, mrep)
    W2p = jnp.pad(W2, ((0, _RP - _R), (0, 0)))
    b2p = jnp.pad(b2, (0, _RP - _R))
    scores = _mlp(sums, W1, b1, W2p, b2p)[:, :_R]
    return scores, emb


# speedup vs baseline: 1.0671x; 1.0671x over previous
import functools

import jax
import jax.numpy as jnp
from jax import lax
from jax.experimental import pallas as pl
from jax.experimental.pallas import tpu as pltpu
from jax.experimental.pallas import tpu_sc as plsc

from jax._src import tpu_custom_call as _tcc

_orig_lowering = _tcc._tpu_custom_call_lowering


def _patched_lowering(ctx, *in_nodes, **kwargs):
    from jax._src.interpreters import mlir as _mlir
    orig_custom_call = _mlir.custom_call

    def custom_call_no_operand_layouts(*a, **kw):
        kw["operand_layouts"] = None
        return orig_custom_call(*a, **kw)

    _mlir.custom_call = custom_call_no_operand_layouts
    try:
        return _orig_lowering(ctx, *in_nodes, **kwargs)
    finally:
        _mlir.custom_call = orig_custom_call


_tcc.mlir.register_lowering(_tcc.tpu_custom_call_p, _patched_lowering,
                            platform="tpu")

_B = 1024
_S = 200
_E = 64
_R = 1000
_RP = 1024

_CH = 50
_CPR = _S // (2 * _CH)
_NW = 32
_ROWS_W = _B // _NW
_HALVES_W = _ROWS_W * 2
_CHUNKS_W = _ROWS_W * 4


def _sc_gather_sum(tpair, chunk_ids):
    mesh = plsc.VectorSubcoreMesh(core_axis_name="c", subcore_axis_name="s")

    nbuf = 4
    dist = 2

    @functools.partial(
        pl.kernel,
        out_type=(
            jax.ShapeDtypeStruct((_B, _S, _E), jnp.float32),
            jax.ShapeDtypeStruct((_B, _E), jnp.float32),
        ),
        mesh=mesh,
        scratch_types=[
            pltpu.VMEM((_CHUNKS_W, 64), jnp.int32),
            pltpu.VMEM((_CHUNKS_W, 64), jnp.int32),
            pltpu.VMEM((_CHUNKS_W * 64,), jnp.float32),
            [pltpu.VMEM((_CH, 128), jnp.float32) for _ in range(nbuf)],
            [pltpu.VMEM((_S, _E), jnp.float32) for _ in range(2)],
            pltpu.VMEM((_ROWS_W, _E), jnp.float32),
            [pltpu.SemaphoreType.DMA for _ in range(nbuf)],
            [pltpu.SemaphoreType.DMA for _ in range(2)],
        ],
        compiler_params=pltpu.CompilerParams(needs_layout_passes=False),
    )
    def k(tpair_hbm, cids_hbm, emb_out, sums_out,
          raw_v, pidx_v, par_v, bufs, cbufs, sums_v, gsems, csems):
        cid = lax.axis_index("c")
        sid = lax.axis_index("s")
        wid = sid * 2 + cid
        base_chunk = wid * _CHUNKS_W
        base_row = wid * _ROWS_W

        pltpu.sync_copy(cids_hbm.at[pl.ds(base_chunk, _CHUNKS_W)], raw_v)
        one = jnp.ones((16,), jnp.float32)

        @pl.loop(0, _CHUNKS_W)
        def derive(ch):
            for j in range(4):
                off = j * 16
                v = raw_v[ch, pl.ds(off, 16)]
                pidx_v[ch, pl.ds(off, 16)] = v >> 1
                par_v[pl.ds(ch * 64 + off, 16)] = (
                    one - (v & 1).astype(jnp.float32))

        zero = jnp.zeros((16,), jnp.float32)

        def chunk_src(c):
            idx = pidx_v.at[c, pl.ds(0, _CH)]
            return tpair_hbm.at[idx]

        def fire_gather(c, b):
            pltpu.async_copy(chunk_src(c), bufs[b], gsems[b])

        def wait_gather(c, b):
            pltpu.make_async_copy(chunk_src(c), bufs[b], gsems[b]).wait()

        iota16 = lax.iota(jnp.int32, 16)

        def acc_blend(buf, c, cbuf, s_base, acc):
            @pl.loop(0, _CH, init_carry=acc, unroll=5)
            def inner(s, acc):
                a0, a1, a2, a3 = acc
                m = plsc.load_gather(par_v, [iota16 * 0 + (c * 64 + s)])
                out = []
                for cc in range(4):
                    va = buf[s, pl.ds(cc * 16, 16)]
                    vb = buf[s, pl.ds(64 + cc * 16, 16)]
                    v = vb + (va - vb) * m
                    cbuf[s_base + s, pl.ds(cc * 16, 16)] = v
                    out.append(v)
                return (a0 + out[0], a1 + out[1], a2 + out[2], a3 + out[3])
            return inner

        for b in range(dist):
            fire_gather(b, b)

        @pl.loop(0, _CHUNKS_W, step=8)
        def outer(c0):
            acc = None
            for b in range(8):
                c = c0 + b
                r = c0 // 4 + b // 4
                cb = b // 4
                bi = b % 4
                wait_gather(c, bi)

                @pl.when(c + dist < _CHUNKS_W)
                def _():
                    fire_gather(c + dist, (bi + dist) % nbuf)

                if b % 4 == 0:
                    @pl.when(r >= 2)
                    def _():
                        pltpu.make_async_copy(
                            cbufs[cb], emb_out.at[base_row + r - 2],
                            csems[cb]).wait()
                    acc = acc_blend(bufs[bi], c, cbufs[cb],
                                    0, (zero, zero, zero, zero))
                else:
                    acc = acc_blend(bufs[bi], c, cbufs[cb],
                                    (b % 4) * _CH, acc)
                if b % 4 == 3:
                    sums_v[r, pl.ds(0, 16)] = acc[0]
                    sums_v[r, pl.ds(16, 16)] = acc[1]
                    sums_v[r, pl.ds(32, 16)] = acc[2]
                    sums_v[r, pl.ds(48, 16)] = acc[3]
                    pltpu.async_copy(cbufs[cb], emb_out.at[base_row + r],
                                     csems[cb])

        for cb in range(2):
            pltpu.make_async_copy(
                cbufs[cb], emb_out.at[base_row + _ROWS_W - 2 + cb],
                csems[cb]).wait()

        pltpu.sync_copy(sums_v, sums_out.at[pl.ds(base_row, _ROWS_W)])

    return k(tpair, chunk_ids)


def _mlp_body(s_ref, w1_ref, b1_ref, w2_ref, b2_ref, o_ref):
    x = s_ref[...] * (1.0 / _S)
    h = lax.dot_general(x, w1_ref[...], (((1,), (1,)), ((), ())),
                        preferred_element_type=jnp.float32) + b1_ref[...]
    h = jnp.maximum(h, 0.0)
    o_ref[...] = lax.dot_general(h, w2_ref[...], (((1,), (1,)), ((), ())),
                                 preferred_element_type=jnp.float32) + b2_ref[...]


def _mlp(sums, W1, b1, W2p, b2p):
    return pl.pallas_call(
        _mlp_body,
        out_shape=jax.ShapeDtypeStruct((_B, _RP), jnp.float32),
    )(sums, W1, b1.reshape(1, _E), W2p, b2p.reshape(1, _RP))


def kernel(atom_ids, atom_table, W1, b1, W2, b2):
    ids = atom_ids.astype(jnp.int32).reshape(4 * _B, _CH)
    chunk_ids = jnp.pad(ids, ((0, 0), (0, 64 - _CH)))
    tpair = atom_table.reshape(500000, 128)
    emb, sums = _sc_gather_sum(tpair, chunk_ids)
    W2p = jnp.pad(W2, ((0, _RP - _R), (0, 0)))
    b2p = jnp.pad(b2, (0, _RP - _R))
    scores = _mlp(sums, W1, b1, W2p, b2p)[:, :_R]
    return scores, emb


# speedup vs baseline: 1.1827x; 1.1084x over previous
import functools

import jax
import jax.numpy as jnp
from jax import lax
from jax.experimental import pallas as pl
from jax.experimental.pallas import tpu as pltpu
from jax.experimental.pallas import tpu_sc as plsc

_B = 1024
_S = 200
_E = 64
_R = 1000
_RP = 1024

_HALF = 100
_NW = 32
_ROWS_W = _B // _NW
_HALVES_W = _ROWS_W * 2


@functools.partial(jax.jit, static_argnames=())
def _sc_gather_sum(table, ids2):
    mesh = plsc.VectorSubcoreMesh(core_axis_name="c", subcore_axis_name="s")

    nbuf = 8
    dist = 4

    @functools.partial(
        pl.kernel,
        out_type=(
            jax.ShapeDtypeStruct((2 * _B, _HALF, _E), jnp.float32),
            jax.ShapeDtypeStruct((_B, _E), jnp.float32),
        ),
        mesh=mesh,
        scratch_types=[
            pltpu.VMEM((_HALVES_W, _HALF), jnp.int32),
            [pltpu.VMEM((_HALF, _E), jnp.float32) for _ in range(nbuf)],
            pltpu.VMEM((_ROWS_W, _E), jnp.float32),
            [pltpu.SemaphoreType.DMA for _ in range(nbuf)],
            [pltpu.SemaphoreType.DMA for _ in range(nbuf)],
        ],
        compiler_params=pltpu.CompilerParams(use_tc_tiling_on_sc=False),
    )
    def k(table_hbm, ids_hbm, emb_out, sums_out, idx_v, bufs, sums_v,
          gsems, ssems):
        cid = lax.axis_index("c")
        sid = lax.axis_index("s")
        wid = sid * 2 + cid
        base_half = wid * _HALVES_W
        base_row = wid * _ROWS_W

        pltpu.sync_copy(ids_hbm.at[pl.ds(base_half, _HALVES_W)], idx_v)

        zero = jnp.zeros((16,), jnp.float32)

        def fire_gather(h, b):
            pltpu.async_copy(table_hbm.at[idx_v.at[h]], bufs[b], gsems[b])

        def acc_half(buf, acc):
            @pl.loop(0, _HALF, init_carry=acc, unroll=4)
            def inner(s, acc):
                a0, a1, a2, a3 = acc
                return (
                    a0 + buf[s, pl.ds(0, 16)],
                    a1 + buf[s, pl.ds(16, 16)],
                    a2 + buf[s, pl.ds(32, 16)],
                    a3 + buf[s, pl.ds(48, 16)],
                )
            return inner

        for b in range(dist):
            fire_gather(b, b)

        @pl.loop(0, _HALVES_W, step=nbuf)
        def outer(g0):
            acc = None
            for b in range(nbuf):
                g = g0 + b
                pltpu.make_async_copy(
                    table_hbm.at[idx_v.at[g]], bufs[b], gsems[b]).wait()
                if b % 2 == 0:
                    acc = acc_half(bufs[b], (zero, zero, zero, zero))
                else:
                    acc = acc_half(bufs[b], acc)
                    r = (g0 + b - 1) // 2
                    sums_v[r, pl.ds(0, 16)] = acc[0]
                    sums_v[r, pl.ds(16, 16)] = acc[1]
                    sums_v[r, pl.ds(32, 16)] = acc[2]
                    sums_v[r, pl.ds(48, 16)] = acc[3]
                pltpu.async_copy(bufs[b], emb_out.at[base_half + g], ssems[b])
                b2 = (b + dist) % nbuf
                g2 = g + dist

                @pl.when(g2 - nbuf >= 0)
                def _():
                    pltpu.make_async_copy(
                        bufs[b2], emb_out.at[base_half + g2 - nbuf],
                        ssems[b2]).wait()

                @pl.when(g2 < _HALVES_W)
                def _():
                    fire_gather(g2, b2)

        for i in range(dist):
            g = _HALVES_W - dist + i
            b = g % nbuf
            pltpu.make_async_copy(
                bufs[b], emb_out.at[base_half + g], ssems[b]).wait()

        pltpu.sync_copy(sums_v, sums_out.at[pl.ds(base_row, _ROWS_W)])

    return k(table, ids2)


def _mlp_body(s_ref, w1_ref, b1_ref, w2_ref, b2_ref, o_ref):
    x = s_ref[...] * (1.0 / _S)
    h = lax.dot_general(x, w1_ref[...], (((1,), (1,)), ((), ())),
                        preferred_element_type=jnp.float32) + b1_ref[...]
    h = jnp.maximum(h, 0.0)
    o_ref[...] = lax.dot_general(h, w2_ref[...], (((1,), (1,)), ((), ())),
                                 preferred_element_type=jnp.float32) + b2_ref[...]


def _mlp(sums, W1, b1, W2p, b2p):
    return pl.pallas_call(
        _mlp_body,
        out_shape=jax.ShapeDtypeStruct((_B, _RP), jnp.float32),
    )(sums, W1, b1.reshape(1, _E), W2p, b2p.reshape(1, _RP))


def kernel(atom_ids, atom_table, W1, b1, W2, b2):
    ids2 = atom_ids.astype(jnp.int32).reshape(2 * _B, _HALF)
    emb2, sums = _sc_gather_sum(atom_table, ids2)
    embeddings = emb2.reshape(_B, _S, _E)
    W2p = jnp.pad(W2, ((0, _RP - _R), (0, 0)))
    b2p = jnp.pad(b2, (0, _RP - _R))
    scores = _mlp(sums, W1, b1, W2p, b2p)[:, :_R]
    return scores, embeddings
